# baseline (speedup 1.0000x reference)
import functools

import jax
import jax.numpy as jnp
from jax import lax
from jax.experimental import pallas as pl
from jax.experimental.pallas import tpu as pltpu
from jax.experimental.pallas import tpu_sc as plsc

_B = 16384
_D = 32
_NC = 2
_NS = 16
_NW = _NC * _NS
_BPW = _B // _NW
_CHUNK = 128
_NCHUNK = _BPW // _CHUNK
_LANES = 16
_GPC = _CHUNK // _LANES


def _gmf_body(uq_hbm, uo_hbm, iq_hbm, io_hbm, ut_hbm, it_hbm, out_hbm,
              uq_v, uo_v, iq_v, io_v, ubuf, ibuf, out_s, sem):
    wid = lax.axis_index("s") * _NC + lax.axis_index("c")

    pltpu.sync_copy(uq_hbm.at[pl.ds(wid * _NCHUNK, _NCHUNK)], uq_v)
    pltpu.sync_copy(uo_hbm.at[pl.ds(wid * _NCHUNK, _NCHUNK)], uo_v)
    pltpu.sync_copy(iq_hbm.at[pl.ds(wid * _NCHUNK, _NCHUNK)], iq_v)
    pltpu.sync_copy(io_hbm.at[pl.ds(wid * _NCHUNK, _NCHUNK)], io_v)

    def fire(j):
        b = j % 2
        return (
            pltpu.async_copy(ut_hbm.at[uq_v.at[j]], ubuf.at[b], sem),
            pltpu.async_copy(it_hbm.at[iq_v.at[j]], ibuf.at[b], sem),
        )

    iota = lax.iota(jnp.int32, _LANES)

    def compute_chunk(j):
        b = j % 2
        ub = ubuf.at[b]
        ib = ibuf.at[b]

        def body(m, carry):
            lrow = iota + m * _LANES
            uo = uo_v[j, pl.ds(m * _LANES, _LANES)]
            io = io_v[j, pl.ds(m * _LANES, _LANES)]
            fbase = (j * _CHUNK + m * _LANES) * _D + iota * _D
            for d in range(_D):
                u = plsc.load_gather(ub, [lrow, uo + d])
                v = plsc.load_gather(ib, [lrow, io + d])
                flat = fbase + d
                plsc.store_scatter(out_s, [flat >> 7, flat & 127], u * v)
            return carry

        lax.fori_loop(0, _GPC, body, 0)

    inflight = fire(0)
    for j in range(_NCHUNK):
        nxt = fire(j + 1) if j + 1 < _NCHUNK else ()
        for c in inflight:
            c.wait()
        compute_chunk(j)
        inflight = nxt

    pltpu.sync_copy(out_s, out_hbm.at[pl.ds(wid * _CHUNK, _CHUNK)])


@functools.partial(
    pl.kernel,
    out_type=jax.ShapeDtypeStruct((_B * _D // 128, 128), jnp.float32),
    mesh=plsc.VectorSubcoreMesh(core_axis_name="c", subcore_axis_name="s"),
    compiler_params=pltpu.CompilerParams(
        use_tc_tiling_on_sc=True, needs_layout_passes=False),
    scratch_types=[
        pltpu.VMEM((_NCHUNK, _CHUNK), jnp.int32),
        pltpu.VMEM((_NCHUNK, _CHUNK), jnp.int32),
        pltpu.VMEM((_NCHUNK, _CHUNK), jnp.int32),
        pltpu.VMEM((_NCHUNK, _CHUNK), jnp.int32),
        pltpu.VMEM((2, _CHUNK, 128), jnp.float32),
        pltpu.VMEM((2, _CHUNK, 128), jnp.float32),
        pltpu.VMEM((_CHUNK, 128), jnp.float32),
        pltpu.SemaphoreType.DMA,
    ],
)
def _gmf(uq_hbm, uo_hbm, iq_hbm, io_hbm, ut_hbm, it_hbm, out_hbm,
         uq_v, uo_v, iq_v, io_v, ubuf, ibuf, out_s, sem):
    _gmf_body(uq_hbm, uo_hbm, iq_hbm, io_hbm, ut_hbm, it_hbm, out_hbm,
              uq_v, uo_v, iq_v, io_v, ubuf, ibuf, out_s, sem)


def kernel(user, item, user_table, item_table):
    user = user.astype(jnp.int32)
    item = item.astype(jnp.int32)
    uq = (user >> 2).reshape(_B // _CHUNK, _CHUNK)
    uo = ((user & 3) * _D).reshape(_B // _CHUNK, _CHUNK)
    iq = (item >> 2).reshape(_B // _CHUNK, _CHUNK)
    io = ((item & 3) * _D).reshape(_B // _CHUNK, _CHUNK)
    ut = user_table.reshape(-1, 128)
    it = item_table.reshape(-1, 128)
    out = _gmf(uq, uo, iq, io, ut, it)
    return out.reshape(_B, _D)


# speedup vs baseline: 4.4718x; 4.4718x over previous
import functools

import jax
import jax.numpy as jnp
from jax import lax
from jax.experimental import pallas as pl
from jax.experimental.pallas import tpu as pltpu
from jax.experimental.pallas import tpu_sc as plsc

_B = 16384
_D = 32
_NC = 2
_NS = 16
_NW = _NC * _NS
_BPW = _B // _NW
_CHUNK = 128
_NCHUNK = _BPW // _CHUNK
_LANES = 16
_RING = 8


def _gmf_body(uix_hbm, iix_hbm, ut_hbm, it_hbm, out_hbm,
              uidx_v, iidx_v, uslab, islab, outT, sems):
    wid = lax.axis_index("s") * _NC + lax.axis_index("c")
    iota = lax.iota(jnp.int32, _LANES)

    pltpu.sync_copy(uix_hbm.at[pl.ds(wid * _NCHUNK, _NCHUNK)], uidx_v)
    pltpu.sync_copy(iix_hbm.at[pl.ds(wid * _NCHUNK, _NCHUNK)], iidx_v)

    def read_idx(ref, i):
        c0 = (i % _CHUNK) >> 4 << 4
        vec = ref[i // _CHUNK, pl.ds(c0, _LANES)]
        lane = i & (_LANES - 1)
        return jnp.sum(jnp.where(iota == lane, vec, 0), axis=0)

    def fire(i, slot):
        ru = read_idx(uidx_v, i)
        ri = read_idx(iidx_v, i)
        ug = pl.multiple_of((ru >> 7) * _CHUNK, _CHUNK)
        ig = pl.multiple_of((ri >> 7) * _CHUNK, _CHUNK)
        pltpu.async_copy(
            ut_hbm.at[:, pl.ds(ug, _CHUNK)], uslab.at[slot], sems[slot])
        pltpu.async_copy(
            it_hbm.at[:, pl.ds(ig, _CHUNK)], islab.at[slot], sems[slot])

    def wait_pair(slot):
        pltpu.make_async_copy(
            ut_hbm.at[:, pl.ds(0, _CHUNK)], uslab.at[slot], sems[slot]).wait()
        pltpu.make_async_copy(
            it_hbm.at[:, pl.ds(0, _CHUNK)], islab.at[slot], sems[slot]).wait()

    def extract(slot, i):
        ru = read_idx(uidx_v, i)
        ri = read_idx(iidx_v, i)
        cu = jnp.full((_LANES,), ru & 127, jnp.int32)
        ci = jnp.full((_LANES,), ri & 127, jnp.int32)
        icol = jnp.full((_LANES,), i, jnp.int32)
        for h in range(2):
            rows = iota + h * _LANES
            u = plsc.load_gather(uslab.at[slot], [rows, cu])
            v = plsc.load_gather(islab.at[slot], [rows, ci])
            plsc.store_scatter(outT, [rows, icol], u * v)

    for p in range(_RING):
        fire(p, p)

    def group(g, carry):
        for p in range(_RING):
            i = g * _RING + p
            wait_pair(p)
            extract(p, i)
            @pl.when(i + _RING < _BPW)
            def _():
                fire(i + _RING, p)
        return carry

    lax.fori_loop(0, _BPW // _RING, group, 0)

    pltpu.sync_copy(outT, out_hbm.at[:, pl.ds(wid * _BPW, _BPW)])


@functools.partial(
    pl.kernel,
    out_type=jax.ShapeDtypeStruct((_D, _B), jnp.float32),
    mesh=plsc.VectorSubcoreMesh(core_axis_name="c", subcore_axis_name="s"),
    compiler_params=pltpu.CompilerParams(
        use_tc_tiling_on_sc=True, needs_layout_passes=False),
    scratch_types=[
        pltpu.VMEM((_NCHUNK, _CHUNK), jnp.int32),
        pltpu.VMEM((_NCHUNK, _CHUNK), jnp.int32),
        pltpu.VMEM((_RING, _D, _CHUNK), jnp.float32),
        pltpu.VMEM((_RING, _D, _CHUNK), jnp.float32),
        pltpu.VMEM((_D, _BPW), jnp.float32),
        [pltpu.SemaphoreType.DMA] * _RING,
    ],
)
def _gmf(uix_hbm, iix_hbm, ut_hbm, it_hbm, out_hbm,
         uidx_v, iidx_v, uslab, islab, outT, sems):
    _gmf_body(uix_hbm, iix_hbm, ut_hbm, it_hbm, out_hbm,
              uidx_v, iidx_v, uslab, islab, outT, sems)


def kernel(user, item, user_table, item_table):
    uix = user.astype(jnp.int32).reshape(_B // _CHUNK, _CHUNK)
    iix = item.astype(jnp.int32).reshape(_B // _CHUNK, _CHUNK)
    out_t = _gmf(uix, iix, user_table.T, item_table.T)
    return out_t.T
